# baseline (speedup 1.0000x reference)
import jax
import jax.numpy as jnp
from jax.experimental import pallas as pl

K = 1024
D = 256
COLS = 512
C = 128


def _vq_block(lat_ref, cb_ref, cbt_ref, eye_ref, out_ref, loss_ref):
    cb = cb_ref[...]
    cbt = cbt_ref[...]
    eye = eye_ref[...]
    cb2 = jnp.sum(cb * cb, axis=1)
    thw = lat_ref.shape[1]
    acc = jnp.zeros((), jnp.float32)
    for t in range(thw // COLS):
        lt = lat_ref[:, t * COLS:(t + 1) * COLS]
        flat = jax.lax.dot_general(eye, lt, (((1,), (1,)), ((), ())),
                                   preferred_element_type=jnp.float32)
        f2 = jnp.sum(flat * flat, axis=1, keepdims=True)
        mm = jax.lax.dot_general(flat, cb, (((1,), (1,)), ((), ())),
                                 preferred_element_type=jnp.float32)
        iota_cf = jax.lax.broadcasted_iota(
            jnp.int32, (COLS, C), 1).astype(jnp.float32)
        val = (f2 + cb2[0:C]) - 2.0 * mm[:, 0:C]
        ind = iota_cf
        for c in range(1, K // C):
            dc = (f2 + cb2[c * C:(c + 1) * C]) - 2.0 * mm[:, c * C:(c + 1) * C]
            lt_ = dc < val
            val = jnp.minimum(val, dc)
            ind = jnp.where(lt_, iota_cf + float(c * C), ind)
        m = jnp.min(val, axis=1, keepdims=True)
        idxf = jnp.min(jnp.where(val == m, ind, float(K)), axis=1,
                       keepdims=True)
        idx = idxf.astype(jnp.int32)
        iota = jax.lax.broadcasted_iota(jnp.int32, (COLS, K), 1)
        oh = (iota == idx).astype(jnp.bfloat16)
        qt = jax.lax.dot_general(cbt, oh, (((1,), (1,)), ((), ())),
                                 preferred_element_type=jnp.float32)
        dt = qt - lt
        out_ref[:, t * COLS:(t + 1) * COLS] = lt + dt
        acc = acc + jnp.sum(dt * dt)
    loss_ref[...] = jnp.full((1, 1, 128), acc, jnp.float32)


def kernel(latents, vq_weight, codebook):
    b, d, t, h, w = latents.shape
    thw = t * h * w
    lat2 = latents.reshape(b * d, thw)
    eye = jnp.eye(COLS, dtype=jnp.bfloat16)
    out2, lossp = pl.pallas_call(
        _vq_block,
        grid=(b,),
        in_specs=[pl.BlockSpec((D, thw), lambda i: (i, 0)),
                  pl.BlockSpec((K, D), lambda i: (0, 0)),
                  pl.BlockSpec((D, K), lambda i: (0, 0)),
                  pl.BlockSpec((COLS, COLS), lambda i: (0, 0))],
        out_specs=[pl.BlockSpec((D, thw), lambda i: (i, 0)),
                   pl.BlockSpec((1, 1, 128), lambda i: (i, 0, 0))],
        out_shape=[jax.ShapeDtypeStruct((b * d, thw), jnp.float32),
                   jax.ShapeDtypeStruct((b, 1, 128), jnp.float32)],
    )(lat2, codebook, codebook.T, eye)
    out3 = out2
    s = jnp.sum(lossp[:, 0, 0])
    mean = s / (b * thw * d)
    vq_loss = mean * vq_weight + mean
    return out3.reshape(b, d, t, h, w), vq_loss


# speedup vs baseline: 4.0083x; 4.0083x over previous
import jax
import jax.numpy as jnp
from jax.experimental import pallas as pl

K = 1024
D = 256
NB = 4096
C = 128


def _vq_block(flat_ref, cb_ref, out_ref, loss_ref):
    flat = flat_ref[...]
    cb = cb_ref[...]
    f2 = jnp.sum(flat * flat, axis=1, keepdims=True)
    cb2 = jnp.sum(cb * cb, axis=1)
    mm = jax.lax.dot_general(flat, cb, (((1,), (1,)), ((), ())),
                             preferred_element_type=jnp.float32)
    iota_cf = jax.lax.broadcasted_iota(
        jnp.int32, (NB, C), 1).astype(jnp.float32)
    val = (f2 + cb2[0:C]) - 2.0 * mm[:, 0:C]
    ind = iota_cf
    for c in range(1, K // C):
        dc = (f2 + cb2[c * C:(c + 1) * C]) - 2.0 * mm[:, c * C:(c + 1) * C]
        lt = dc < val
        val = jnp.minimum(val, dc)
        ind = jnp.where(lt, iota_cf + float(c * C), ind)
    m = jnp.min(val, axis=1, keepdims=True)
    idxf = jnp.min(jnp.where(val == m, ind, float(K)), axis=1, keepdims=True)
    idx = idxf.astype(jnp.int32)
    iota = jax.lax.broadcasted_iota(jnp.int32, (NB, K), 1)
    oh = (iota == idx).astype(jnp.bfloat16)
    q = jax.lax.dot_general(oh, cb, (((1,), (0,)), ((), ())),
                            preferred_element_type=jnp.float32)
    diff = q - flat
    out_ref[...] = flat + diff
    loss_ref[...] = jnp.full((1, 1, 128), jnp.sum(diff * diff), jnp.float32)


def kernel(latents, vq_weight, codebook):
    lat = jnp.transpose(latents, (0, 2, 3, 4, 1))
    lat_shape = lat.shape
    flat = lat.reshape(-1, D)
    n = flat.shape[0]
    nblk = n // NB
    out, lossp = pl.pallas_call(
        _vq_block,
        grid=(nblk,),
        in_specs=[pl.BlockSpec((NB, D), lambda i: (i, 0)),
                  pl.BlockSpec((K, D), lambda i: (0, 0))],
        out_specs=[pl.BlockSpec((NB, D), lambda i: (i, 0)),
                   pl.BlockSpec((1, 1, 128), lambda i: (i, 0, 0))],
        out_shape=[jax.ShapeDtypeStruct((n, D), jnp.float32),
                   jax.ShapeDtypeStruct((nblk, 1, 128), jnp.float32)],
    )(flat, codebook)
    s = jnp.sum(lossp[:, 0, 0])
    mean = s / (n * D)
    vq_loss = mean * vq_weight + mean
    out5 = out.reshape(lat_shape)
    return jnp.transpose(out5, (0, 4, 1, 2, 3)), vq_loss
